# baseline (speedup 1.0000x reference)
import functools

import jax
import jax.numpy as jnp
from jax import lax
from jax.experimental import pallas as pl
from jax.experimental.pallas import tpu as pltpu
from jax.experimental.pallas import tpu_sc as plsc

B = 256
NN, NE, NC = 50000, 100000, 20000
MAXN, MAXE, MAXC = 256, 512, 128
D = 64
L = 3
OUT = 10
NSUB = 16
NW = 32
CH = 1024
PCH = 2048

f32 = jnp.float32
i32 = jnp.int32


def _ceil_to(x, m):
    return (x + m - 1) // m * m


def _make_edge_kernel(E_pad, ND, two_tables, relu):
    big = ND > 60000
    CH = (320 if two_tables else 512) if big else 1024
    ZB = 128
    eper = E_pad // NSUB
    NCH = -(-eper // CH)
    NDP = ND + 16
    zr = NDP // 16
    wr = ND // 16
    mesh = plsc.VectorSubcoreMesh(core_axis_name="c", subcore_axis_name="s")

    scratch = [
        pltpu.VMEM((4, CH), i32),
        pltpu.VMEM((3, CH), i32),
        pltpu.VMEM((2, CH, 16), f32),
        pltpu.VMEM((ZB, 16), f32),
        pltpu.VMEM_SHARED((NDP, 16), f32),
    ]
    nsem = 4 + 3 + 2 + 2 + 1
    if two_tables:
        scratch += [
            pltpu.VMEM((3, CH), i32),
            pltpu.VMEM((2, CH, 16), f32),
        ]
        nsem += 3 + 2
    scratch += [pltpu.SemaphoreType.DMA] * nsem

    @functools.partial(
        pl.kernel,
        out_type=jax.ShapeDtypeStruct((ND, 4, 16), f32),
        mesh=mesh,
        name=f"edge{2 if two_tables else 1}_E{E_pad}_D{ND}",
        scratch_types=scratch,
        compiler_params=pltpu.CompilerParams(use_tc_tiling_on_sc=False),
    )
    def k(*refs):
        if two_tables:
            (dst, s1, s2, t1, t2, out, dbuf, s1buf, rows1, zbuf, acc,
             s2buf, rows2, *sems) = refs
        else:
            (dst, s1, t1, out, dbuf, s1buf, rows1, zbuf, acc, *sems) = refs
        dsem = sems[0:4]
        s1sem = sems[4:7]
        g1sem = sems[7:9]
        scsem = sems[9:11]
        zsem = sems[11]
        if two_tables:
            s2sem = sems[12:15]
            g2sem = sems[15:17]
        cid = lax.axis_index("c")
        sid = lax.axis_index("s")

        def zb(i, _):
            zbuf[i, :] = jnp.zeros((16,), f32)
            return 0
        lax.fori_loop(0, ZB, zb, 0)

        idx_d = {}

        def issue_idx(kk):
            base = sid * eper + kk * CH
            n = min(CH, eper - kk * CH)
            c1 = pltpu.async_copy(dst.at[pl.ds(base, n)],
                                  dbuf.at[kk % 4, pl.ds(0, n)], dsem[kk % 4])
            c2 = pltpu.async_copy(s1.at[pl.ds(base, n)],
                                  s1buf.at[kk % 3, pl.ds(0, n)],
                                  s1sem[kk % 3])
            cs = [c1, c2]
            if two_tables:
                cs.append(pltpu.async_copy(s2.at[pl.ds(base, n)],
                                           s2buf.at[kk % 3, pl.ds(0, n)],
                                           s2sem[kk % 3]))
            idx_d[kk] = cs

        def transform(kk, cp):
            b3, b4 = kk % 3, kk % 4
            n = min(CH, eper - kk * CH)

            def tr(i, _):
                s1buf[b3, pl.ds(i * 16, 16)] = (
                    s1buf[b3, pl.ds(i * 16, 16)] * 4 + cp)
                if two_tables:
                    s2buf[b3, pl.ds(i * 16, 16)] = (
                        s2buf[b3, pl.ds(i * 16, 16)] * 4 + cp)
                return 0
            lax.fori_loop(0, CH // 16, tr, 0)
            if n < CH:
                def pd(i, _):
                    dbuf[b4, pl.ds(n + i * 16, 16)] = jnp.full((16,), ND, i32)
                    s1buf[b3, pl.ds(n + i * 16, 16)] = jnp.full((16,), cp, i32)
                    if two_tables:
                        s2buf[b3, pl.ds(n + i * 16, 16)] = jnp.full(
                            (16,), cp, i32)
                    return 0
                lax.fori_loop(0, (CH - n) // 16, pd, 0)

        for cg in range(2):
            cp = cid * 2 + cg
            zc = []
            nz = -(-zr // ZB)
            for zi in range(nz):
                w = min(ZB, zr - zi * ZB)
                zc.append(pltpu.async_copy(
                    zbuf.at[pl.ds(0, w), :],
                    acc.at[pl.ds(sid * zr + zi * ZB, w), :], zsem))
            for c in zc:
                c.wait()
            plsc.subcore_barrier()

            sc_d = {}
            issue_idx(0)
            issue_idx(1)
            for c in idx_d[0]:
                c.wait()
            transform(0, cp)
            for kk in range(NCH):
                b2, b3, b4 = kk % 2, kk % 3, kk % 4
                if kk >= 2:
                    sc_d[kk - 2].wait()
                g1 = pltpu.async_copy(t1.at[s1buf.at[b3]], rows1.at[b2],
                                      g1sem[b2])
                if two_tables:
                    g2 = pltpu.async_copy(t2.at[s2buf.at[b3]], rows2.at[b2],
                                          g2sem[b2])
                if kk + 1 < NCH:
                    for c in idx_d[kk + 1]:
                        c.wait()
                    transform(kk + 1, cp)
                if kk + 2 < NCH:
                    issue_idx(kk + 2)
                g1.wait()
                if two_tables:
                    g2.wait()

                    def fuse(r, _):
                        v = rows1[b2, r, :] + rows2[b2, r, :]
                        if relu:
                            v = jnp.maximum(v, 0.0)
                        rows1[b2, r, :] = v
                        return 0
                    lax.fori_loop(0, CH, fuse, 0)
                sc_d[kk] = pltpu.async_copy(rows1.at[b2], acc.at[dbuf.at[b4]],
                                            scsem[b2], add=True)
            sc_d[NCH - 1].wait()
            if NCH >= 2:
                sc_d[NCH - 2].wait()
            plsc.subcore_barrier()

            wc = []
            for zi in range(-(-wr // ZB)):
                w = min(ZB, wr - zi * ZB)
                r0 = sid * wr + zi * ZB
                wc.append(pltpu.async_copy(acc.at[pl.ds(r0, w), :],
                                           out.at[pl.ds(r0, w), cp, :], zsem))
            for c in wc:
                c.wait()
            plsc.subcore_barrier()

    return k


_edge2_relu = {}
_edge1 = {}


def _edge_pass2(dst, s1, s2, t1, t2, ND):
    E = dst.shape[0]
    EP = _ceil_to(E, 256)
    if EP != E:
        dst = jnp.pad(dst, (0, EP - E), constant_values=ND)
        s1 = jnp.pad(s1, (0, EP - E))
        s2 = jnp.pad(s2, (0, EP - E))
    key = (EP, ND)
    if key not in _edge2_relu:
        _edge2_relu[key] = _make_edge_kernel(EP, ND, True, True)
    out = _edge2_relu[key](dst, s1, s2,
                           t1.reshape(-1, 16), t2.reshape(-1, 16))
    return out.reshape(ND, 64)


def _edge_pass1(dst, s1, t1, ND):
    E = dst.shape[0]
    EP = _ceil_to(E, 256)
    if EP != E:
        dst = jnp.pad(dst, (0, EP - E), constant_values=ND)
        s1 = jnp.pad(s1, (0, EP - E))
    key = (EP, ND)
    if key not in _edge1:
        _edge1[key] = _make_edge_kernel(EP, ND, False, False)
    out = _edge1[key](dst, s1, t1.reshape(-1, 16))
    return out.reshape(ND, 64)


def _make_scan_kernel(NP, mx):
    bpw = B // NW
    nchunks = NP // PCH
    mesh = plsc.VectorSubcoreMesh(core_axis_name="c", subcore_axis_name="s")

    @functools.partial(
        pl.kernel,
        out_type=jax.ShapeDtypeStruct((NW, bpw * mx), i32),
        mesh=mesh,
        name=f"scan_mx{mx}",
        compiler_params=pltpu.CompilerParams(use_tc_tiling_on_sc=False,
                                             needs_layout_passes=False),
        scratch_types=[
            pltpu.VMEM((2, PCH), i32),
            pltpu.VMEM((2, PCH), i32),
            pltpu.VMEM((bpw * mx,), i32),
            pltpu.VMEM((32,), i32),
            pltpu.SemaphoreType.DMA,
            pltpu.SemaphoreType.DMA,
        ],
    )
    def k(bt_hbm, pos_hbm, out, btbuf, posbuf, win, kbuf, sem0, sem1):
        cid = lax.axis_index("c")
        sid = lax.axis_index("s")
        wid = cid * NSUB + sid
        lo = wid * bpw
        sems = [sem0, sem1]

        def wi(i, _):
            win[pl.ds(i * 16, 16)] = jnp.full((16,), -1, i32)
            return 0
        lax.fori_loop(0, bpw * mx // 16, wi, 0)
        kbuf[pl.ds(16, 16)] = jnp.full((16,), -1, i32)

        lds = {}

        def issue(ch):
            b = ch % 2
            lds[ch] = (
                pltpu.async_copy(bt_hbm.at[pl.ds(ch * PCH, PCH)],
                                 btbuf.at[b], sems[b]),
                pltpu.async_copy(pos_hbm.at[pl.ds(ch * PCH, PCH)],
                                 posbuf.at[b], sems[b]),
            )

        MAXK = jnp.int32(0x7FFFFFFF)
        issue(0)
        for ch in range(nchunks):
            b = ch % 2
            for c in lds[ch]:
                c.wait()
            if ch + 1 < nchunks:
                issue(ch + 1)

            def scan(i, _):
                btv = btbuf[b, pl.ds(i * 16, 16)]
                posv = posbuf[b, pl.ds(i * 16, 16)]
                inr = (btv >= lo) & (btv < lo + bpw)
                some = jnp.max(plsc.all_reduce_population_count(inr)) > 0

                @pl.when(some)
                def _slow():
                    slot = (btv - lo) * mx + posv
                    item = lax.iota(i32, 16) + (ch * PCH + i * 16)
                    key = jnp.where(inr, slot * 131072 + item, MAXK)
                    ks, _vs = plsc.sort_key_val(key, key)
                    kbuf[pl.ds(0, 16)] = ks
                    nxt = plsc.load_gather(kbuf, [lax.iota(i32, 16) + 1])
                    slot_s = lax.shift_right_logical(ks, 17)
                    run_end = slot_s != lax.shift_right_logical(nxt, 17)
                    ok = run_end & (ks != MAXK)
                    plsc.store_scatter(win, [jnp.where(ok, slot_s, 0)],
                                       ks & 131071, mask=ok)
                return 0
            lax.fori_loop(0, PCH // 16, scan, 0)
        pltpu.sync_copy(win, out.at[wid])

    return k


def _make_reduce_kernel(N, mx):
    bpw = B // NW
    mesh = plsc.VectorSubcoreMesh(core_axis_name="c", subcore_axis_name="s")

    @functools.partial(
        pl.kernel,
        out_type=jax.ShapeDtypeStruct((B, 64), f32),
        mesh=mesh,
        name=f"reduce_N{N}",
        compiler_params=pltpu.CompilerParams(use_tc_tiling_on_sc=False),
        scratch_types=[
            pltpu.VMEM((bpw * mx,), i32),
            pltpu.VMEM((2, mx, 64), f32),
            pltpu.VMEM((bpw, 64), f32),
            pltpu.SemaphoreType.DMA,
            pltpu.SemaphoreType.DMA,
        ],
    )
    def k(win_hbm, xg, out, gidx, rows, obuf, sem0, sem1):
        cid = lax.axis_index("c")
        sid = lax.axis_index("s")
        wid = cid * NSUB + sid
        lo = wid * bpw
        sems = [sem0, sem1]

        pltpu.sync_copy(win_hbm.at[wid], gidx)

        def tr(i, _):
            v = gidx[pl.ds(i * 16, 16)]
            gidx[pl.ds(i * 16, 16)] = jnp.where(v >= 0, v, N)
            return 0
        lax.fori_loop(0, bpw * mx // 16, tr, 0)

        gds = {}

        def issue(bb):
            gds[bb] = pltpu.async_copy(
                xg.at[gidx.at[pl.ds(bb * mx, mx)]], rows.at[bb % 2],
                sems[bb % 2])

        issue(0)
        for bb in range(bpw):
            r2 = bb % 2
            if bb + 1 < bpw:
                issue(bb + 1)
            gds[bb].wait()

            def red(r, a):
                return (a[0] + rows[r2, r, pl.ds(0, 16)],
                        a[1] + rows[r2, r, pl.ds(16, 16)],
                        a[2] + rows[r2, r, pl.ds(32, 16)],
                        a[3] + rows[r2, r, pl.ds(48, 16)])
            z = jnp.zeros((16,), f32)
            a0, a1, a2, a3 = lax.fori_loop(0, mx, red, (z, z, z, z))
            obuf[bb, pl.ds(0, 16)] = a0
            obuf[bb, pl.ds(16, 16)] = a1
            obuf[bb, pl.ds(32, 16)] = a2
            obuf[bb, pl.ds(48, 16)] = a3
        pltpu.sync_copy(obuf, out.at[pl.ds(lo, bpw), :])

    return k


_scan = {}
_reduce = {}


def _pool_pass(bt, pos, xg_pad, N, mx):
    NP = _ceil_to(N, PCH)
    if NP != N:
        bt = jnp.pad(bt, (0, NP - N), constant_values=B)
        pos = jnp.pad(pos, (0, NP - N))
    if (NP, mx) not in _scan:
        _scan[(NP, mx)] = _make_scan_kernel(NP, mx)
    if (N, mx) not in _reduce:
        _reduce[(N, mx)] = _make_reduce_kernel(N, mx)
    win = _scan[(NP, mx)](bt, pos)
    return _reduce[(N, mx)](win, xg_pad)


BN = 400


def _embed_n_body(t_ref, emb_ref, w_ref, x_ref, p_ref):
    oh = (t_ref[0, 0, :][:, None] == lax.broadcasted_iota(i32, (BN, 128), 1))
    x = jnp.dot(oh.astype(f32), emb_ref[:], preferred_element_type=f32)
    x_ref[:] = x
    p_ref[:] = jnp.dot(x, w_ref[:], preferred_element_type=f32)


def _embed_e_body(t_ref, emb_ref, w1_ref, w2_ref, x_ref, q_ref, r_ref):
    oh = (t_ref[0, 0, :][:, None] == lax.broadcasted_iota(i32, (BN, 128), 1))
    x = jnp.dot(oh.astype(f32), emb_ref[:], preferred_element_type=f32)
    x_ref[:] = x
    q_ref[:] = jnp.dot(x, w1_ref[:], preferred_element_type=f32)
    r_ref[:] = jnp.dot(x, w2_ref[:], preferred_element_type=f32)


def _row_spec():
    return pl.BlockSpec((BN, 64), lambda i: (i, 0))


def _full_spec(shape):
    nd = len(shape)
    return pl.BlockSpec(shape, lambda i: (0,) * nd)


def _embed_n(types, emb_pad, W):
    N = types.shape[0]
    types = types.reshape(N // BN, 1, BN)
    return pl.pallas_call(
        _embed_n_body,
        grid=(N // BN,),
        in_specs=[pl.BlockSpec((1, 1, BN), lambda i: (i, 0, 0)),
                  _full_spec((128, 64)), _full_spec((64, 64))],
        out_specs=[_row_spec(), _row_spec()],
        out_shape=[jax.ShapeDtypeStruct((N, 64), f32)] * 2,
    )(types, emb_pad, W)


def _embed_e(types, emb_pad, W1, W2):
    N = types.shape[0]
    types = types.reshape(N // BN, 1, BN)
    return pl.pallas_call(
        _embed_e_body,
        grid=(N // BN,),
        in_specs=[pl.BlockSpec((1, 1, BN), lambda i: (i, 0, 0)),
                  _full_spec((128, 64)), _full_spec((64, 64)),
                  _full_spec((64, 64))],
        out_specs=[_row_spec()] * 3,
        out_shape=[jax.ShapeDtypeStruct((N, 64), f32)] * 3,
    )(types, emb_pad, W1, W2)


def _proj_body(x_ref, w_ref, o_ref):
    o_ref[:] = jnp.dot(x_ref[:], w_ref[:], preferred_element_type=f32)


def _proj(x, W):
    N = x.shape[0]
    return pl.pallas_call(
        _proj_body,
        grid=(N // BN,),
        in_specs=[_row_spec(), _full_spec((64, 64))],
        out_specs=_row_spec(),
        out_shape=jax.ShapeDtypeStruct((N, 64), f32),
    )(x, W)


def _upd_n_body(x_ref, a_ref, wu_ref, wn_ref, xo_ref, po_ref):
    xn = jnp.maximum(jnp.dot(x_ref[:] + a_ref[:], wu_ref[:],
                             preferred_element_type=f32), 0.0)
    xo_ref[:] = xn
    po_ref[:] = jnp.dot(xn, wn_ref[:], preferred_element_type=f32)


def _upd_n(x, agg, Wupd, Wnext):
    N = x.shape[0]
    return pl.pallas_call(
        _upd_n_body,
        grid=(N // BN,),
        in_specs=[_row_spec(), _row_spec(), _full_spec((64, 64)),
                  _full_spec((64, 64))],
        out_specs=[_row_spec(), _row_spec()],
        out_shape=[jax.ShapeDtypeStruct((N, 64), f32)] * 2,
    )(x, agg, Wupd, Wnext)


def _upd_e_body(x_ref, a_ref, t_ref, wu_ref, wb_ref, w1_ref, w2_ref,
                xo_ref, qo_ref, ro_ref):
    w2c = jnp.dot(wb_ref[:], wu_ref[:], preferred_element_type=f32)
    xe = jnp.maximum(
        jnp.dot(x_ref[:] + a_ref[:], wu_ref[:], preferred_element_type=f32)
        + jnp.dot(t_ref[:], w2c, preferred_element_type=f32), 0.0)
    xo_ref[:] = xe
    qo_ref[:] = jnp.dot(xe, w1_ref[:], preferred_element_type=f32)
    ro_ref[:] = jnp.dot(xe, w2_ref[:], preferred_element_type=f32)


def _upd_e(x, agg, T, Wupd, Wbd, Wn1, Wn2):
    N = x.shape[0]
    return pl.pallas_call(
        _upd_e_body,
        grid=(N // BN,),
        in_specs=[_row_spec(), _row_spec(), _row_spec()]
        + [_full_spec((64, 64))] * 4,
        out_specs=[_row_spec()] * 3,
        out_shape=[jax.ShapeDtypeStruct((N, 64), f32)] * 3,
    )(x, agg, T, Wupd, Wbd, Wn1, Wn2)


def _upd_c_body(x_ref, u_ref, wu_ref, wb_ref, wn_ref, xo_ref, so_ref):
    w2c = jnp.dot(wb_ref[:], wu_ref[:], preferred_element_type=f32)
    xc = jnp.maximum(
        jnp.dot(x_ref[:], wu_ref[:], preferred_element_type=f32)
        + jnp.dot(u_ref[:], w2c, preferred_element_type=f32), 0.0)
    xo_ref[:] = xc
    so_ref[:] = jnp.dot(xc, wn_ref[:], preferred_element_type=f32)


def _upd_c(x, U, Wupd, Wbd, Wnext):
    N = x.shape[0]
    return pl.pallas_call(
        _upd_c_body,
        grid=(N // BN,),
        in_specs=[_row_spec(), _row_spec()] + [_full_spec((64, 64))] * 3,
        out_specs=[_row_spec(), _row_spec()],
        out_shape=[jax.ShapeDtypeStruct((N, 64), f32)] * 2,
    )(x, U, Wupd, Wbd, Wnext)


def _updgate_body(nblk, upd_fn, refs):
    i = pl.program_id(0)
    out_ref = refs[-1]

    @pl.when(i == nblk)
    def _zero():
        out_ref[:] = jnp.zeros((BN, 64), f32)

    @pl.when(i < nblk)
    def _compute():
        out_ref[:] = upd_fn(*refs[:-1])


def _gate_n_fn(x_ref, a_ref, r0_ref, wu_ref, wp0_ref, wp1_ref):
    xn = jnp.maximum(jnp.dot(x_ref[:] + a_ref[:], wu_ref[:],
                             preferred_element_type=f32), 0.0)
    g = jax.nn.sigmoid(jnp.dot(xn, wp0_ref[:], preferred_element_type=f32)
                       + jnp.dot(r0_ref[:], wp1_ref[:],
                                 preferred_element_type=f32))
    return g * xn


def _gate_e_fn(x_ref, a_ref, t_ref, r0_ref, wu_ref, wb_ref, wp0_ref, wp1_ref):
    w2c = jnp.dot(wb_ref[:], wu_ref[:], preferred_element_type=f32)
    xe = jnp.maximum(
        jnp.dot(x_ref[:] + a_ref[:], wu_ref[:], preferred_element_type=f32)
        + jnp.dot(t_ref[:], w2c, preferred_element_type=f32), 0.0)
    g = jax.nn.sigmoid(jnp.dot(xe, wp0_ref[:], preferred_element_type=f32)
                       + jnp.dot(r0_ref[:], wp1_ref[:],
                                 preferred_element_type=f32))
    return g * xe


def _gate_c_fn(x_ref, u_ref, r0_ref, wu_ref, wb_ref, wp0_ref, wp1_ref):
    w2c = jnp.dot(wb_ref[:], wu_ref[:], preferred_element_type=f32)
    xc = jnp.maximum(
        jnp.dot(x_ref[:], wu_ref[:], preferred_element_type=f32)
        + jnp.dot(u_ref[:], w2c, preferred_element_type=f32), 0.0)
    g = jax.nn.sigmoid(jnp.dot(xc, wp0_ref[:], preferred_element_type=f32)
                       + jnp.dot(r0_ref[:], wp1_ref[:],
                                 preferred_element_type=f32))
    return g * xc


def _updgate(fn, row_args, mat_args, N):
    nblk = N // BN
    grid = (nblk + 1,)
    clamp = lambda i: (jnp.minimum(i, nblk - 1), 0)
    in_specs = ([pl.BlockSpec((BN, 64), clamp)] * len(row_args)
                + [_full_spec((64, 64))] * len(mat_args))
    body = functools.partial(_updgate_body, nblk,
                             lambda *r: fn(*r))
    return pl.pallas_call(
        lambda *refs: body(refs),
        grid=grid,
        in_specs=in_specs,
        out_specs=_row_spec(),
        out_shape=jax.ShapeDtypeStruct((N + BN, 64), f32),
    )(*row_args, *mat_args)


def _head_body(pn_ref, pe_ref, pc_ref, w1_ref, b1_ref, w2_ref, b2_ref, o_ref):
    acc = jnp.zeros((B, OUT), f32)
    for i, p in enumerate((pn_ref, pe_ref, pc_ref)):
        h = jnp.maximum(jnp.dot(p[:], w1_ref[i], preferred_element_type=f32)
                        + b1_ref[i][None, :], 0.0)
        acc = acc + jnp.dot(h, w2_ref[pl.ds(i * 128, 128), :],
                            preferred_element_type=f32)
    o_ref[:] = acc + b2_ref[:][None, :]


def _head(pn, pe, pc, w1, b1, w2, b2):
    return pl.pallas_call(
        _head_body,
        in_specs=[pl.BlockSpec((B, 64), lambda: (0, 0))] * 3
        + [pl.BlockSpec((3, 64, 128), lambda: (0, 0, 0)),
           pl.BlockSpec((3, 128), lambda: (0, 0)),
           pl.BlockSpec((384, OUT), lambda: (0, 0)),
           pl.BlockSpec((OUT,), lambda: (0,))],
        out_specs=pl.BlockSpec((B, OUT), lambda: (0, 0)),
        out_shape=jax.ShapeDtypeStruct((B, OUT), f32),
    )(pn, pe, pc, w1, b1, w2, b2)


def kernel(atom_type, bond_type, n_up_i, n_up_j, n_up_attr_idx,
           e_up_i, e_up_j, e_up_attr_idx, e_bd_cell, e_bd_node,
           c_bd_cell, c_bd_edge, n_batch, n_pos, e_batch, e_pos,
           c_batch, c_pos, atom_emb, bond_emb, Wn_up, Wn_upd,
           We_up, We_bd, We_upd, Wc_bd, Wc_upd, Wpool,
           lin1_w, lin1_b, lin2_w, lin2_b):
    atom_pad = jnp.pad(atom_emb, ((0, 128 - atom_emb.shape[0]), (0, 0)))
    bond_pad = jnp.pad(bond_emb, ((0, 128 - bond_emb.shape[0]), (0, 0)))

    xn, Pn = _embed_n(atom_type, atom_pad, Wn_up[0])
    xe, Qe, Re = _embed_e(bond_type, bond_pad, Wn_up[0], We_up[0])
    xc = _edge_pass1(c_bd_cell, c_bd_edge, xe, NC)
    Sc = _proj(xc, We_up[0])
    xn0, xe0, xc0 = xn, xe, xc

    gated = [None, None, None]
    for l in range(L):
        agg_n = _edge_pass2(n_up_i, n_up_j, n_up_attr_idx, Pn, Qe, NN)
        agg_e = _edge_pass2(e_up_i, e_up_j, e_up_attr_idx, Re, Sc, NE)
        T = _edge_pass1(e_bd_cell, e_bd_node, xn, NE)
        U = _edge_pass1(c_bd_cell, c_bd_edge, xe, NC)
        if l < L - 1:
            xn, Pn = _upd_n(xn, agg_n, Wn_upd[l], Wn_up[l + 1])
            xe, Qe, Re = _upd_e(xe, agg_e, T, We_upd[l], We_bd[l],
                                Wn_up[l + 1], We_up[l + 1])
            xc, Sc = _upd_c(xc, U, Wc_upd[l], Wc_bd[l], We_up[l + 1])
        else:
            gated[0] = _updgate(_gate_n_fn, (xn, agg_n, xn0),
                                (Wn_upd[l], Wpool[0, 0], Wpool[0, 1]), NN)
            gated[1] = _updgate(_gate_e_fn, (xe, agg_e, T, xe0),
                                (We_upd[l], We_bd[l], Wpool[1, 0],
                                 Wpool[1, 1]), NE)
            gated[2] = _updgate(_gate_c_fn, (xc, U, xc0),
                                (Wc_upd[l], Wc_bd[l], Wpool[2, 0],
                                 Wpool[2, 1]), NC)

    pn = _pool_pass(n_batch, n_pos, gated[0], NN, MAXN)
    pe = _pool_pass(e_batch, e_pos, gated[1], NE, MAXE)
    pc = _pool_pass(c_batch, c_pos, gated[2], NC, MAXC)
    return _head(pn, pe, pc, lin1_w, lin1_b, lin2_w, lin2_b)


# speedup vs baseline: 1.0209x; 1.0209x over previous
import functools

import jax
import jax.numpy as jnp
from jax import lax
from jax.experimental import pallas as pl
from jax.experimental.pallas import tpu as pltpu
from jax.experimental.pallas import tpu_sc as plsc

B = 256
NN, NE, NC = 50000, 100000, 20000
MAXN, MAXE, MAXC = 256, 512, 128
D = 64
L = 3
OUT = 10
NSUB = 16
NW = 32
CH = 1024
PCH = 2048

f32 = jnp.float32
i32 = jnp.int32


def _ceil_to(x, m):
    return (x + m - 1) // m * m


def _make_edge_kernel(E_pad, ND, two_tables, relu):
    big = ND > 60000
    CH = (320 if two_tables else 512) if big else 1024
    ZB = 128
    eper = E_pad // NSUB
    NCH = -(-eper // CH)
    NDP = ND + 16
    zr = NDP // 16
    wr = ND // 16
    mesh = plsc.VectorSubcoreMesh(core_axis_name="c", subcore_axis_name="s")

    scratch = [
        pltpu.VMEM((4, CH), i32),
        pltpu.VMEM((3, CH), i32),
        pltpu.VMEM((2, CH, 16), f32),
        pltpu.VMEM((ZB, 16), f32),
        pltpu.VMEM_SHARED((NDP, 16), f32),
    ]
    nsem = 4 + 3 + 2 + 2 + 1
    if two_tables:
        scratch += [
            pltpu.VMEM((3, CH), i32),
            pltpu.VMEM((2, CH, 16), f32),
        ]
        nsem += 3 + 2
    scratch += [pltpu.SemaphoreType.DMA] * nsem

    @functools.partial(
        pl.kernel,
        out_type=jax.ShapeDtypeStruct((ND, 4, 16), f32),
        mesh=mesh,
        name=f"edge{2 if two_tables else 1}_E{E_pad}_D{ND}",
        scratch_types=scratch,
        compiler_params=pltpu.CompilerParams(use_tc_tiling_on_sc=False),
    )
    def k(*refs):
        if two_tables:
            (dst, s1, s2, t1, t2, out, dbuf, s1buf, rows1, zbuf, acc,
             s2buf, rows2, *sems) = refs
        else:
            (dst, s1, t1, out, dbuf, s1buf, rows1, zbuf, acc, *sems) = refs
        dsem = sems[0:4]
        s1sem = sems[4:7]
        g1sem = sems[7:9]
        scsem = sems[9:11]
        zsem = sems[11]
        if two_tables:
            s2sem = sems[12:15]
            g2sem = sems[15:17]
        cid = lax.axis_index("c")
        sid = lax.axis_index("s")

        def zb(i, _):
            zbuf[i, :] = jnp.zeros((16,), f32)
            return 0
        lax.fori_loop(0, ZB, zb, 0)

        idx_d = {}

        def issue_idx(kk):
            base = sid * eper + kk * CH
            n = min(CH, eper - kk * CH)
            c1 = pltpu.async_copy(dst.at[pl.ds(base, n)],
                                  dbuf.at[kk % 4, pl.ds(0, n)], dsem[kk % 4])
            c2 = pltpu.async_copy(s1.at[pl.ds(base, n)],
                                  s1buf.at[kk % 3, pl.ds(0, n)],
                                  s1sem[kk % 3])
            cs = [c1, c2]
            if two_tables:
                cs.append(pltpu.async_copy(s2.at[pl.ds(base, n)],
                                           s2buf.at[kk % 3, pl.ds(0, n)],
                                           s2sem[kk % 3]))
            idx_d[kk] = cs

        def transform(kk, cp):
            b3, b4 = kk % 3, kk % 4
            n = min(CH, eper - kk * CH)

            def tr(i, _):
                s1buf[b3, pl.ds(i * 16, 16)] = (
                    s1buf[b3, pl.ds(i * 16, 16)] * 4 + cp)
                if two_tables:
                    s2buf[b3, pl.ds(i * 16, 16)] = (
                        s2buf[b3, pl.ds(i * 16, 16)] * 4 + cp)
                return 0
            lax.fori_loop(0, CH // 16, tr, 0)
            if n < CH:
                def pd(i, _):
                    dbuf[b4, pl.ds(n + i * 16, 16)] = jnp.full((16,), ND, i32)
                    s1buf[b3, pl.ds(n + i * 16, 16)] = jnp.full((16,), cp, i32)
                    if two_tables:
                        s2buf[b3, pl.ds(n + i * 16, 16)] = jnp.full(
                            (16,), cp, i32)
                    return 0
                lax.fori_loop(0, (CH - n) // 16, pd, 0)

        for cg in range(2):
            cp = cid * 2 + cg
            zc = []
            nz = -(-zr // ZB)
            for zi in range(nz):
                w = min(ZB, zr - zi * ZB)
                zc.append(pltpu.async_copy(
                    zbuf.at[pl.ds(0, w), :],
                    acc.at[pl.ds(sid * zr + zi * ZB, w), :], zsem))
            for c in zc:
                c.wait()
            plsc.subcore_barrier()

            sc_d = {}
            issue_idx(0)
            issue_idx(1)
            for c in idx_d[0]:
                c.wait()
            transform(0, cp)
            for kk in range(NCH):
                b2, b3, b4 = kk % 2, kk % 3, kk % 4
                if kk >= 2:
                    sc_d[kk - 2].wait()
                g1 = pltpu.async_copy(t1.at[s1buf.at[b3]], rows1.at[b2],
                                      g1sem[b2])
                if two_tables:
                    g2 = pltpu.async_copy(t2.at[s2buf.at[b3]], rows2.at[b2],
                                          g2sem[b2])
                if kk + 1 < NCH:
                    for c in idx_d[kk + 1]:
                        c.wait()
                    transform(kk + 1, cp)
                if kk + 2 < NCH:
                    issue_idx(kk + 2)
                g1.wait()
                if two_tables:
                    g2.wait()

                    def fuse(r, _):
                        v = rows1[b2, r, :] + rows2[b2, r, :]
                        if relu:
                            v = jnp.maximum(v, 0.0)
                        rows1[b2, r, :] = v
                        return 0
                    lax.fori_loop(0, CH, fuse, 0)
                sc_d[kk] = pltpu.async_copy(rows1.at[b2], acc.at[dbuf.at[b4]],
                                            scsem[b2], add=True)
            sc_d[NCH - 1].wait()
            if NCH >= 2:
                sc_d[NCH - 2].wait()
            plsc.subcore_barrier()

            wc = []
            for zi in range(-(-wr // ZB)):
                w = min(ZB, wr - zi * ZB)
                r0 = sid * wr + zi * ZB
                wc.append(pltpu.async_copy(acc.at[pl.ds(r0, w), :],
                                           out.at[pl.ds(r0, w), cp, :], zsem))
            for c in wc:
                c.wait()
            plsc.subcore_barrier()

    return k


_edge2_relu = {}
_edge1 = {}


def _edge_pass2(dst, s1, s2, t1, t2, ND):
    E = dst.shape[0]
    EP = _ceil_to(E, 256)
    if EP != E:
        dst = jnp.pad(dst, (0, EP - E), constant_values=ND)
        s1 = jnp.pad(s1, (0, EP - E))
        s2 = jnp.pad(s2, (0, EP - E))
    key = (EP, ND)
    if key not in _edge2_relu:
        _edge2_relu[key] = _make_edge_kernel(EP, ND, True, True)
    out = _edge2_relu[key](dst, s1, s2,
                           t1.reshape(-1, 16), t2.reshape(-1, 16))
    return out.reshape(ND, 64)


def _edge_pass1(dst, s1, t1, ND):
    E = dst.shape[0]
    EP = _ceil_to(E, 256)
    if EP != E:
        dst = jnp.pad(dst, (0, EP - E), constant_values=ND)
        s1 = jnp.pad(s1, (0, EP - E))
    key = (EP, ND)
    if key not in _edge1:
        _edge1[key] = _make_edge_kernel(EP, ND, False, False)
    out = _edge1[key](dst, s1, t1.reshape(-1, 16))
    return out.reshape(ND, 64)


def _make_scan_kernel(NP, mx):
    bpw = B // NW
    nchunks = NP // PCH
    mesh = plsc.VectorSubcoreMesh(core_axis_name="c", subcore_axis_name="s")

    @functools.partial(
        pl.kernel,
        out_type=jax.ShapeDtypeStruct((NW, bpw * mx), i32),
        mesh=mesh,
        name=f"scan_mx{mx}",
        compiler_params=pltpu.CompilerParams(use_tc_tiling_on_sc=False,
                                             needs_layout_passes=False),
        scratch_types=[
            pltpu.VMEM((2, PCH), i32),
            pltpu.VMEM((2, PCH), i32),
            pltpu.VMEM((bpw * mx,), i32),
            pltpu.VMEM((32,), i32),
            pltpu.SemaphoreType.DMA,
            pltpu.SemaphoreType.DMA,
        ],
    )
    def k(bt_hbm, pos_hbm, out, btbuf, posbuf, win, kbuf, sem0, sem1):
        cid = lax.axis_index("c")
        sid = lax.axis_index("s")
        wid = cid * NSUB + sid
        lo = wid * bpw
        sems = [sem0, sem1]

        def wi(i, _):
            win[pl.ds(i * 16, 16)] = jnp.full((16,), -1, i32)
            return 0
        lax.fori_loop(0, bpw * mx // 16, wi, 0)
        kbuf[pl.ds(16, 16)] = jnp.full((16,), -1, i32)

        lds = {}

        def issue(ch):
            b = ch % 2
            lds[ch] = (
                pltpu.async_copy(bt_hbm.at[pl.ds(ch * PCH, PCH)],
                                 btbuf.at[b], sems[b]),
                pltpu.async_copy(pos_hbm.at[pl.ds(ch * PCH, PCH)],
                                 posbuf.at[b], sems[b]),
            )

        MAXK = jnp.int32(0x7FFFFFFF)
        issue(0)
        for ch in range(nchunks):
            b = ch % 2
            for c in lds[ch]:
                c.wait()
            if ch + 1 < nchunks:
                issue(ch + 1)

            cmin = jnp.max(-btbuf[b, pl.ds(0, 16)]) * -1
            cmax = jnp.max(btbuf[b, pl.ds(PCH - 16, 16)])
            hit = (cmax >= lo) & (cmin < lo + bpw)

            @pl.when(hit)
            def _do_chunk():
                def scan(i, _):
                    btv = btbuf[b, pl.ds(i * 16, 16)]
                    posv = posbuf[b, pl.ds(i * 16, 16)]
                    inr = (btv >= lo) & (btv < lo + bpw)
                    some = jnp.max(plsc.all_reduce_population_count(inr)) > 0

                    @pl.when(some)
                    def _slow():
                        slot = (btv - lo) * mx + posv
                        item = lax.iota(i32, 16) + (ch * PCH + i * 16)
                        key = jnp.where(inr, slot * 131072 + item, MAXK)
                        ks, _vs = plsc.sort_key_val(key, key)
                        kbuf[pl.ds(0, 16)] = ks
                        nxt = plsc.load_gather(kbuf, [lax.iota(i32, 16) + 1])
                        slot_s = lax.shift_right_logical(ks, 17)
                        run_end = slot_s != lax.shift_right_logical(nxt, 17)
                        ok = run_end & (ks != MAXK)
                        plsc.store_scatter(win, [jnp.where(ok, slot_s, 0)],
                                           ks & 131071, mask=ok)
                    return 0
                lax.fori_loop(0, PCH // 16, scan, 0)
        pltpu.sync_copy(win, out.at[wid])

    return k


def _make_reduce_kernel(N, mx):
    bpw = B // NW
    mesh = plsc.VectorSubcoreMesh(core_axis_name="c", subcore_axis_name="s")

    @functools.partial(
        pl.kernel,
        out_type=jax.ShapeDtypeStruct((B, 64), f32),
        mesh=mesh,
        name=f"reduce_N{N}",
        compiler_params=pltpu.CompilerParams(use_tc_tiling_on_sc=False),
        scratch_types=[
            pltpu.VMEM((bpw, mx), i32),
            pltpu.VMEM((2, mx, 64), f32),
            pltpu.VMEM((bpw, 64), f32),
            pltpu.SemaphoreType.DMA,
            pltpu.SemaphoreType.DMA,
        ],
    )
    def k(win_hbm, xg, out, gidx, rows, obuf, sem0, sem1):
        cid = lax.axis_index("c")
        sid = lax.axis_index("s")
        wid = cid * NSUB + sid
        lo = wid * bpw
        sems = [sem0, sem1]

        for bb in range(bpw):
            pltpu.sync_copy(win_hbm.at[wid, pl.ds(bb * mx, mx)], gidx.at[bb])

        for bb in range(bpw):
            def tr(i, _):
                v = gidx[bb, pl.ds(i * 16, 16)]
                gidx[bb, pl.ds(i * 16, 16)] = jnp.where(v >= 0, v, N)
                return 0
            lax.fori_loop(0, mx // 16, tr, 0)

        gds = {}

        def issue(bb):
            gds[bb] = pltpu.async_copy(
                xg.at[gidx.at[bb]], rows.at[bb % 2], sems[bb % 2])

        issue(0)
        for bb in range(bpw):
            r2 = bb % 2
            if bb + 1 < bpw:
                issue(bb + 1)
            gds[bb].wait()

            def red(r, a):
                r0 = 2 * r
                return (a[0] + rows[r2, r0, pl.ds(0, 16)],
                        a[1] + rows[r2, r0, pl.ds(16, 16)],
                        a[2] + rows[r2, r0, pl.ds(32, 16)],
                        a[3] + rows[r2, r0, pl.ds(48, 16)],
                        a[4] + rows[r2, r0 + 1, pl.ds(0, 16)],
                        a[5] + rows[r2, r0 + 1, pl.ds(16, 16)],
                        a[6] + rows[r2, r0 + 1, pl.ds(32, 16)],
                        a[7] + rows[r2, r0 + 1, pl.ds(48, 16)])
            z = jnp.zeros((16,), f32)
            a = lax.fori_loop(0, mx // 2, red, (z,) * 8)
            obuf[bb, pl.ds(0, 16)] = a[0] + a[4]
            obuf[bb, pl.ds(16, 16)] = a[1] + a[5]
            obuf[bb, pl.ds(32, 16)] = a[2] + a[6]
            obuf[bb, pl.ds(48, 16)] = a[3] + a[7]
        pltpu.sync_copy(obuf, out.at[pl.ds(lo, bpw), :])

    return k


_scan = {}
_reduce = {}


def _pool_pass(bt, pos, xg_pad, N, mx):
    NP = _ceil_to(N, PCH)
    if NP != N:
        bt = jnp.pad(bt, (0, NP - N), constant_values=B)
        pos = jnp.pad(pos, (0, NP - N))
    if (NP, mx) not in _scan:
        _scan[(NP, mx)] = _make_scan_kernel(NP, mx)
    if (N, mx) not in _reduce:
        _reduce[(N, mx)] = _make_reduce_kernel(N, mx)
    win = _scan[(NP, mx)](bt, pos)
    return _reduce[(N, mx)](win, xg_pad)


BN = 400


def _embed_n_body(t_ref, emb_ref, w_ref, x_ref, p_ref):
    oh = (t_ref[0, 0, :][:, None] == lax.broadcasted_iota(i32, (BN, 128), 1))
    x = jnp.dot(oh.astype(f32), emb_ref[:], preferred_element_type=f32)
    x_ref[:] = x
    p_ref[:] = jnp.dot(x, w_ref[:], preferred_element_type=f32)


def _embed_e_body(t_ref, emb_ref, w1_ref, w2_ref, x_ref, q_ref, r_ref):
    oh = (t_ref[0, 0, :][:, None] == lax.broadcasted_iota(i32, (BN, 128), 1))
    x = jnp.dot(oh.astype(f32), emb_ref[:], preferred_element_type=f32)
    x_ref[:] = x
    q_ref[:] = jnp.dot(x, w1_ref[:], preferred_element_type=f32)
    r_ref[:] = jnp.dot(x, w2_ref[:], preferred_element_type=f32)


def _row_spec():
    return pl.BlockSpec((BN, 64), lambda i: (i, 0))


def _full_spec(shape):
    nd = len(shape)
    return pl.BlockSpec(shape, lambda i: (0,) * nd)


def _embed_n(types, emb_pad, W):
    N = types.shape[0]
    types = types.reshape(N // BN, 1, BN)
    return pl.pallas_call(
        _embed_n_body,
        grid=(N // BN,),
        in_specs=[pl.BlockSpec((1, 1, BN), lambda i: (i, 0, 0)),
                  _full_spec((128, 64)), _full_spec((64, 64))],
        out_specs=[_row_spec(), _row_spec()],
        out_shape=[jax.ShapeDtypeStruct((N, 64), f32)] * 2,
    )(types, emb_pad, W)


def _embed_e(types, emb_pad, W1, W2):
    N = types.shape[0]
    types = types.reshape(N // BN, 1, BN)
    return pl.pallas_call(
        _embed_e_body,
        grid=(N // BN,),
        in_specs=[pl.BlockSpec((1, 1, BN), lambda i: (i, 0, 0)),
                  _full_spec((128, 64)), _full_spec((64, 64)),
                  _full_spec((64, 64))],
        out_specs=[_row_spec()] * 3,
        out_shape=[jax.ShapeDtypeStruct((N, 64), f32)] * 3,
    )(types, emb_pad, W1, W2)


def _proj_body(x_ref, w_ref, o_ref):
    o_ref[:] = jnp.dot(x_ref[:], w_ref[:], preferred_element_type=f32)


def _proj(x, W):
    N = x.shape[0]
    return pl.pallas_call(
        _proj_body,
        grid=(N // BN,),
        in_specs=[_row_spec(), _full_spec((64, 64))],
        out_specs=_row_spec(),
        out_shape=jax.ShapeDtypeStruct((N, 64), f32),
    )(x, W)


def _upd_n_body(x_ref, a_ref, wu_ref, wn_ref, xo_ref, po_ref):
    xn = jnp.maximum(jnp.dot(x_ref[:] + a_ref[:], wu_ref[:],
                             preferred_element_type=f32), 0.0)
    xo_ref[:] = xn
    po_ref[:] = jnp.dot(xn, wn_ref[:], preferred_element_type=f32)


def _upd_n(x, agg, Wupd, Wnext):
    N = x.shape[0]
    return pl.pallas_call(
        _upd_n_body,
        grid=(N // BN,),
        in_specs=[_row_spec(), _row_spec(), _full_spec((64, 64)),
                  _full_spec((64, 64))],
        out_specs=[_row_spec(), _row_spec()],
        out_shape=[jax.ShapeDtypeStruct((N, 64), f32)] * 2,
    )(x, agg, Wupd, Wnext)


def _upd_e_body(x_ref, a_ref, t_ref, wu_ref, wb_ref, w1_ref, w2_ref,
                xo_ref, qo_ref, ro_ref):
    w2c = jnp.dot(wb_ref[:], wu_ref[:], preferred_element_type=f32)
    xe = jnp.maximum(
        jnp.dot(x_ref[:] + a_ref[:], wu_ref[:], preferred_element_type=f32)
        + jnp.dot(t_ref[:], w2c, preferred_element_type=f32), 0.0)
    xo_ref[:] = xe
    qo_ref[:] = jnp.dot(xe, w1_ref[:], preferred_element_type=f32)
    ro_ref[:] = jnp.dot(xe, w2_ref[:], preferred_element_type=f32)


def _upd_e(x, agg, T, Wupd, Wbd, Wn1, Wn2):
    N = x.shape[0]
    return pl.pallas_call(
        _upd_e_body,
        grid=(N // BN,),
        in_specs=[_row_spec(), _row_spec(), _row_spec()]
        + [_full_spec((64, 64))] * 4,
        out_specs=[_row_spec()] * 3,
        out_shape=[jax.ShapeDtypeStruct((N, 64), f32)] * 3,
    )(x, agg, T, Wupd, Wbd, Wn1, Wn2)


def _upd_c_body(x_ref, u_ref, wu_ref, wb_ref, wn_ref, xo_ref, so_ref):
    w2c = jnp.dot(wb_ref[:], wu_ref[:], preferred_element_type=f32)
    xc = jnp.maximum(
        jnp.dot(x_ref[:], wu_ref[:], preferred_element_type=f32)
        + jnp.dot(u_ref[:], w2c, preferred_element_type=f32), 0.0)
    xo_ref[:] = xc
    so_ref[:] = jnp.dot(xc, wn_ref[:], preferred_element_type=f32)


def _upd_c(x, U, Wupd, Wbd, Wnext):
    N = x.shape[0]
    return pl.pallas_call(
        _upd_c_body,
        grid=(N // BN,),
        in_specs=[_row_spec(), _row_spec()] + [_full_spec((64, 64))] * 3,
        out_specs=[_row_spec(), _row_spec()],
        out_shape=[jax.ShapeDtypeStruct((N, 64), f32)] * 2,
    )(x, U, Wupd, Wbd, Wnext)


def _updgate_body(nblk, upd_fn, refs):
    i = pl.program_id(0)
    out_ref = refs[-1]

    @pl.when(i == nblk)
    def _zero():
        out_ref[:] = jnp.zeros((BN, 64), f32)

    @pl.when(i < nblk)
    def _compute():
        out_ref[:] = upd_fn(*refs[:-1])


def _gate_n_fn(x_ref, a_ref, r0_ref, wu_ref, wp0_ref, wp1_ref):
    xn = jnp.maximum(jnp.dot(x_ref[:] + a_ref[:], wu_ref[:],
                             preferred_element_type=f32), 0.0)
    g = jax.nn.sigmoid(jnp.dot(xn, wp0_ref[:], preferred_element_type=f32)
                       + jnp.dot(r0_ref[:], wp1_ref[:],
                                 preferred_element_type=f32))
    return g * xn


def _gate_e_fn(x_ref, a_ref, t_ref, r0_ref, wu_ref, wb_ref, wp0_ref, wp1_ref):
    w2c = jnp.dot(wb_ref[:], wu_ref[:], preferred_element_type=f32)
    xe = jnp.maximum(
        jnp.dot(x_ref[:] + a_ref[:], wu_ref[:], preferred_element_type=f32)
        + jnp.dot(t_ref[:], w2c, preferred_element_type=f32), 0.0)
    g = jax.nn.sigmoid(jnp.dot(xe, wp0_ref[:], preferred_element_type=f32)
                       + jnp.dot(r0_ref[:], wp1_ref[:],
                                 preferred_element_type=f32))
    return g * xe


def _gate_c_fn(x_ref, u_ref, r0_ref, wu_ref, wb_ref, wp0_ref, wp1_ref):
    w2c = jnp.dot(wb_ref[:], wu_ref[:], preferred_element_type=f32)
    xc = jnp.maximum(
        jnp.dot(x_ref[:], wu_ref[:], preferred_element_type=f32)
        + jnp.dot(u_ref[:], w2c, preferred_element_type=f32), 0.0)
    g = jax.nn.sigmoid(jnp.dot(xc, wp0_ref[:], preferred_element_type=f32)
                       + jnp.dot(r0_ref[:], wp1_ref[:],
                                 preferred_element_type=f32))
    return g * xc


def _updgate(fn, row_args, mat_args, N):
    nblk = N // BN
    grid = (nblk + 1,)
    clamp = lambda i: (jnp.minimum(i, nblk - 1), 0)
    in_specs = ([pl.BlockSpec((BN, 64), clamp)] * len(row_args)
                + [_full_spec((64, 64))] * len(mat_args))
    body = functools.partial(_updgate_body, nblk,
                             lambda *r: fn(*r))
    return pl.pallas_call(
        lambda *refs: body(refs),
        grid=grid,
        in_specs=in_specs,
        out_specs=_row_spec(),
        out_shape=jax.ShapeDtypeStruct((N + BN, 64), f32),
    )(*row_args, *mat_args)


def _head_body(pn_ref, pe_ref, pc_ref, w1_ref, b1_ref, w2_ref, b2_ref, o_ref):
    acc = jnp.zeros((B, OUT), f32)
    for i, p in enumerate((pn_ref, pe_ref, pc_ref)):
        h = jnp.maximum(jnp.dot(p[:], w1_ref[i], preferred_element_type=f32)
                        + b1_ref[i][None, :], 0.0)
        acc = acc + jnp.dot(h, w2_ref[pl.ds(i * 128, 128), :],
                            preferred_element_type=f32)
    o_ref[:] = acc + b2_ref[:][None, :]


def _head(pn, pe, pc, w1, b1, w2, b2):
    return pl.pallas_call(
        _head_body,
        in_specs=[pl.BlockSpec((B, 64), lambda: (0, 0))] * 3
        + [pl.BlockSpec((3, 64, 128), lambda: (0, 0, 0)),
           pl.BlockSpec((3, 128), lambda: (0, 0)),
           pl.BlockSpec((384, OUT), lambda: (0, 0)),
           pl.BlockSpec((OUT,), lambda: (0,))],
        out_specs=pl.BlockSpec((B, OUT), lambda: (0, 0)),
        out_shape=jax.ShapeDtypeStruct((B, OUT), f32),
    )(pn, pe, pc, w1, b1, w2, b2)


def kernel(atom_type, bond_type, n_up_i, n_up_j, n_up_attr_idx,
           e_up_i, e_up_j, e_up_attr_idx, e_bd_cell, e_bd_node,
           c_bd_cell, c_bd_edge, n_batch, n_pos, e_batch, e_pos,
           c_batch, c_pos, atom_emb, bond_emb, Wn_up, Wn_upd,
           We_up, We_bd, We_upd, Wc_bd, Wc_upd, Wpool,
           lin1_w, lin1_b, lin2_w, lin2_b):
    atom_pad = jnp.pad(atom_emb, ((0, 128 - atom_emb.shape[0]), (0, 0)))
    bond_pad = jnp.pad(bond_emb, ((0, 128 - bond_emb.shape[0]), (0, 0)))

    xn, Pn = _embed_n(atom_type, atom_pad, Wn_up[0])
    xe, Qe, Re = _embed_e(bond_type, bond_pad, Wn_up[0], We_up[0])
    xc = _edge_pass1(c_bd_cell, c_bd_edge, xe, NC)
    Sc = _proj(xc, We_up[0])
    xn0, xe0, xc0 = xn, xe, xc

    gated = [None, None, None]
    for l in range(L):
        agg_n = _edge_pass2(n_up_i, n_up_j, n_up_attr_idx, Pn, Qe, NN)
        agg_e = _edge_pass2(e_up_i, e_up_j, e_up_attr_idx, Re, Sc, NE)
        T = _edge_pass1(e_bd_cell, e_bd_node, xn, NE)
        U = _edge_pass1(c_bd_cell, c_bd_edge, xe, NC)
        if l < L - 1:
            xn, Pn = _upd_n(xn, agg_n, Wn_upd[l], Wn_up[l + 1])
            xe, Qe, Re = _upd_e(xe, agg_e, T, We_upd[l], We_bd[l],
                                Wn_up[l + 1], We_up[l + 1])
            xc, Sc = _upd_c(xc, U, Wc_upd[l], Wc_bd[l], We_up[l + 1])
        else:
            gated[0] = _updgate(_gate_n_fn, (xn, agg_n, xn0),
                                (Wn_upd[l], Wpool[0, 0], Wpool[0, 1]), NN)
            gated[1] = _updgate(_gate_e_fn, (xe, agg_e, T, xe0),
                                (We_upd[l], We_bd[l], Wpool[1, 0],
                                 Wpool[1, 1]), NE)
            gated[2] = _updgate(_gate_c_fn, (xc, U, xc0),
                                (Wc_upd[l], Wc_bd[l], Wpool[2, 0],
                                 Wpool[2, 1]), NC)

    pn = _pool_pass(n_batch, n_pos, gated[0], NN, MAXN)
    pe = _pool_pass(e_batch, e_pos, gated[1], NE, MAXE)
    pc = _pool_pass(c_batch, c_pos, gated[2], NC, MAXC)
    return _head(pn, pe, pc, lin1_w, lin1_b, lin2_w, lin2_b)


# speedup vs baseline: 1.0235x; 1.0025x over previous
import functools

import jax
import jax.numpy as jnp
from jax import lax
from jax.experimental import pallas as pl
from jax.experimental.pallas import tpu as pltpu
from jax.experimental.pallas import tpu_sc as plsc

B = 256
NN, NE, NC = 50000, 100000, 20000
MAXN, MAXE, MAXC = 256, 512, 128
D = 64
L = 3
OUT = 10
NSUB = 16
NW = 32
CH = 1024
PCH = 2048

f32 = jnp.float32
i32 = jnp.int32


def _ceil_to(x, m):
    return (x + m - 1) // m * m


def _make_edge_kernel(E_pad, ND, two_tables, relu):
    big = ND > 60000
    CH = (320 if two_tables else 512) if big else 1024
    ZB = 128
    eper = E_pad // NSUB
    NCH = -(-eper // CH)
    NDP = ND + 16
    zr = NDP // 16
    wr = ND // 16
    mesh = plsc.VectorSubcoreMesh(core_axis_name="c", subcore_axis_name="s")

    scratch = [
        pltpu.VMEM((4, CH), i32),
        pltpu.VMEM((3, CH), i32),
        pltpu.VMEM((2, CH, 16), f32),
        pltpu.VMEM((ZB, 16), f32),
        pltpu.VMEM_SHARED((NDP, 16), f32),
    ]
    nsem = 4 + 3 + 2 + 2 + 1
    if two_tables:
        scratch += [
            pltpu.VMEM((3, CH), i32),
            pltpu.VMEM((2, CH, 16), f32),
        ]
        nsem += 3 + 2
    scratch += [pltpu.SemaphoreType.DMA] * nsem

    @functools.partial(
        pl.kernel,
        out_type=jax.ShapeDtypeStruct((ND, 4, 16), f32),
        mesh=mesh,
        name=f"edge{2 if two_tables else 1}_E{E_pad}_D{ND}",
        scratch_types=scratch,
        compiler_params=pltpu.CompilerParams(use_tc_tiling_on_sc=False),
    )
    def k(*refs):
        if two_tables:
            (dst, s1, s2, t1, t2, out, dbuf, s1buf, rows1, zbuf, acc,
             s2buf, rows2, *sems) = refs
        else:
            (dst, s1, t1, out, dbuf, s1buf, rows1, zbuf, acc, *sems) = refs
        dsem = sems[0:4]
        s1sem = sems[4:7]
        g1sem = sems[7:9]
        scsem = sems[9:11]
        zsem = sems[11]
        if two_tables:
            s2sem = sems[12:15]
            g2sem = sems[15:17]
        cid = lax.axis_index("c")
        sid = lax.axis_index("s")

        def zb(i, _):
            zbuf[i, :] = jnp.zeros((16,), f32)
            return 0
        lax.fori_loop(0, ZB, zb, 0)

        idx_d = {}

        def issue_idx(kk):
            base = sid * eper + kk * CH
            n = min(CH, eper - kk * CH)
            c1 = pltpu.async_copy(dst.at[pl.ds(base, n)],
                                  dbuf.at[kk % 4, pl.ds(0, n)], dsem[kk % 4])
            c2 = pltpu.async_copy(s1.at[pl.ds(base, n)],
                                  s1buf.at[kk % 3, pl.ds(0, n)],
                                  s1sem[kk % 3])
            cs = [c1, c2]
            if two_tables:
                cs.append(pltpu.async_copy(s2.at[pl.ds(base, n)],
                                           s2buf.at[kk % 3, pl.ds(0, n)],
                                           s2sem[kk % 3]))
            idx_d[kk] = cs

        def transform(kk, cp):
            b3, b4 = kk % 3, kk % 4
            n = min(CH, eper - kk * CH)

            def tr(i, _):
                s1buf[b3, pl.ds(i * 16, 16)] = (
                    s1buf[b3, pl.ds(i * 16, 16)] * 4 + cp)
                if two_tables:
                    s2buf[b3, pl.ds(i * 16, 16)] = (
                        s2buf[b3, pl.ds(i * 16, 16)] * 4 + cp)
                return 0
            lax.fori_loop(0, CH // 16, tr, 0)
            if n < CH:
                def pd(i, _):
                    dbuf[b4, pl.ds(n + i * 16, 16)] = jnp.full((16,), ND, i32)
                    s1buf[b3, pl.ds(n + i * 16, 16)] = jnp.full((16,), cp, i32)
                    if two_tables:
                        s2buf[b3, pl.ds(n + i * 16, 16)] = jnp.full(
                            (16,), cp, i32)
                    return 0
                lax.fori_loop(0, (CH - n) // 16, pd, 0)

        for cg in range(2):
            cp = cid * 2 + cg
            zc = []
            nz = -(-zr // ZB)
            for zi in range(nz):
                w = min(ZB, zr - zi * ZB)
                zc.append(pltpu.async_copy(
                    zbuf.at[pl.ds(0, w), :],
                    acc.at[pl.ds(sid * zr + zi * ZB, w), :], zsem))
            for c in zc:
                c.wait()
            plsc.subcore_barrier()

            sc_d = {}
            issue_idx(0)
            issue_idx(1)
            for c in idx_d[0]:
                c.wait()
            transform(0, cp)
            for kk in range(NCH):
                b2, b3, b4 = kk % 2, kk % 3, kk % 4
                if kk >= 2:
                    sc_d[kk - 2].wait()
                g1 = pltpu.async_copy(t1.at[s1buf.at[b3]], rows1.at[b2],
                                      g1sem[b2])
                if two_tables:
                    g2 = pltpu.async_copy(t2.at[s2buf.at[b3]], rows2.at[b2],
                                          g2sem[b2])
                if kk + 1 < NCH:
                    for c in idx_d[kk + 1]:
                        c.wait()
                    transform(kk + 1, cp)
                if kk + 2 < NCH:
                    issue_idx(kk + 2)
                g1.wait()
                if two_tables:
                    g2.wait()

                    def fuse(r, _):
                        v = rows1[b2, r, :] + rows2[b2, r, :]
                        if relu:
                            v = jnp.maximum(v, 0.0)
                        rows1[b2, r, :] = v
                        return 0
                    lax.fori_loop(0, CH, fuse, 0)
                sc_d[kk] = pltpu.async_copy(rows1.at[b2], acc.at[dbuf.at[b4]],
                                            scsem[b2], add=True)
            sc_d[NCH - 1].wait()
            if NCH >= 2:
                sc_d[NCH - 2].wait()
            plsc.subcore_barrier()

            wc = []
            for zi in range(-(-wr // ZB)):
                w = min(ZB, wr - zi * ZB)
                r0 = sid * wr + zi * ZB
                wc.append(pltpu.async_copy(acc.at[pl.ds(r0, w), :],
                                           out.at[pl.ds(r0, w), cp, :], zsem))
            for c in wc:
                c.wait()
            plsc.subcore_barrier()

    return k


_edge2_relu = {}
_edge1 = {}


def _edge_pass2(dst, s1, s2, t1, t2, ND):
    E = dst.shape[0]
    EP = _ceil_to(E, 256)
    if EP != E:
        dst = jnp.pad(dst, (0, EP - E), constant_values=ND)
        s1 = jnp.pad(s1, (0, EP - E))
        s2 = jnp.pad(s2, (0, EP - E))
    key = (EP, ND)
    if key not in _edge2_relu:
        _edge2_relu[key] = _make_edge_kernel(EP, ND, True, True)
    out = _edge2_relu[key](dst, s1, s2,
                           t1.reshape(-1, 16), t2.reshape(-1, 16))
    return out.reshape(ND, 64)


def _edge_pass1(dst, s1, t1, ND):
    E = dst.shape[0]
    EP = _ceil_to(E, 256)
    if EP != E:
        dst = jnp.pad(dst, (0, EP - E), constant_values=ND)
        s1 = jnp.pad(s1, (0, EP - E))
    key = (EP, ND)
    if key not in _edge1:
        _edge1[key] = _make_edge_kernel(EP, ND, False, False)
    out = _edge1[key](dst, s1, t1.reshape(-1, 16))
    return out.reshape(ND, 64)


def _make_scan_kernel(NP, mx, N):
    bpw = B // NW
    nchunks = NP // PCH
    mesh = plsc.VectorSubcoreMesh(core_axis_name="c", subcore_axis_name="s")

    @functools.partial(
        pl.kernel,
        out_type=jax.ShapeDtypeStruct((NW, bpw * mx), i32),
        mesh=mesh,
        name=f"scan_mx{mx}",
        compiler_params=pltpu.CompilerParams(use_tc_tiling_on_sc=False,
                                             needs_layout_passes=False),
        scratch_types=[
            pltpu.VMEM((2, PCH), i32),
            pltpu.VMEM((2, PCH), i32),
            pltpu.VMEM((bpw * mx,), i32),
            pltpu.VMEM((32,), i32),
            pltpu.SemaphoreType.DMA,
            pltpu.SemaphoreType.DMA,
        ],
    )
    def k(bt_hbm, pos_hbm, out, btbuf, posbuf, win, kbuf, sem0, sem1):
        cid = lax.axis_index("c")
        sid = lax.axis_index("s")
        wid = cid * NSUB + sid
        lo = wid * bpw
        sems = [sem0, sem1]

        def wi(i, _):
            win[pl.ds(i * 16, 16)] = jnp.full((16,), -1, i32)
            return 0
        lax.fori_loop(0, bpw * mx // 16, wi, 0)
        kbuf[pl.ds(16, 16)] = jnp.full((16,), -1, i32)

        lds = {}

        def issue(ch):
            b = ch % 2
            lds[ch] = (
                pltpu.async_copy(bt_hbm.at[pl.ds(ch * PCH, PCH)],
                                 btbuf.at[b], sems[b]),
                pltpu.async_copy(pos_hbm.at[pl.ds(ch * PCH, PCH)],
                                 posbuf.at[b], sems[b]),
            )

        MAXK = jnp.int32(0x7FFFFFFF)
        issue(0)
        for ch in range(nchunks):
            b = ch % 2
            for c in lds[ch]:
                c.wait()
            if ch + 1 < nchunks:
                issue(ch + 1)

            cmin = jnp.max(-btbuf[b, pl.ds(0, 16)]) * -1
            cmax = jnp.max(btbuf[b, pl.ds(PCH - 16, 16)])
            hit = (cmax >= lo) & (cmin < lo + bpw)

            @pl.when(hit)
            def _do_chunk():
                def scan(i, _):
                    btv = btbuf[b, pl.ds(i * 16, 16)]
                    posv = posbuf[b, pl.ds(i * 16, 16)]
                    inr = (btv >= lo) & (btv < lo + bpw)
                    some = jnp.max(plsc.all_reduce_population_count(inr)) > 0

                    @pl.when(some)
                    def _slow():
                        slot = (btv - lo) * mx + posv
                        item = lax.iota(i32, 16) + (ch * PCH + i * 16)
                        key = jnp.where(inr, slot * 131072 + item, MAXK)
                        ks, _vs = plsc.sort_key_val(key, key)
                        kbuf[pl.ds(0, 16)] = ks
                        nxt = plsc.load_gather(kbuf, [lax.iota(i32, 16) + 1])
                        slot_s = lax.shift_right_logical(ks, 17)
                        run_end = slot_s != lax.shift_right_logical(nxt, 17)
                        ok = run_end & (ks != MAXK)
                        plsc.store_scatter(win, [jnp.where(ok, slot_s, 0)],
                                           ks & 131071, mask=ok)
                    return 0
                lax.fori_loop(0, PCH // 16, scan, 0)

        def fin(i, _):
            v = win[pl.ds(i * 16, 16)]
            win[pl.ds(i * 16, 16)] = jnp.where(v >= 0, v, N)
            return 0
        lax.fori_loop(0, bpw * mx // 16, fin, 0)
        pltpu.sync_copy(win, out.at[wid])

    return k


_scan = {}


def _pool_pass(bt, pos, xg_pad, N, mx):
    NP = _ceil_to(N, PCH)
    if NP != N:
        bt = jnp.pad(bt, (0, NP - N), constant_values=B)
        pos = jnp.pad(pos, (0, NP - N))
    if (NP, mx, N) not in _scan:
        _scan[(NP, mx, N)] = _make_scan_kernel(NP, mx, N)
    win = _scan[(NP, mx, N)](bt, pos).reshape(-1)
    slot_batch = jnp.arange(B * mx, dtype=i32) // mx
    return _edge_pass1(slot_batch, win, xg_pad, B)


BN = 400


def _embed_n_body(t_ref, emb_ref, w_ref, x_ref, p_ref):
    oh = (t_ref[0, 0, :][:, None] == lax.broadcasted_iota(i32, (BN, 128), 1))
    x = jnp.dot(oh.astype(f32), emb_ref[:], preferred_element_type=f32)
    x_ref[:] = x
    p_ref[:] = jnp.dot(x, w_ref[:], preferred_element_type=f32)


def _embed_e_body(t_ref, emb_ref, w1_ref, w2_ref, x_ref, q_ref, r_ref):
    oh = (t_ref[0, 0, :][:, None] == lax.broadcasted_iota(i32, (BN, 128), 1))
    x = jnp.dot(oh.astype(f32), emb_ref[:], preferred_element_type=f32)
    x_ref[:] = x
    q_ref[:] = jnp.dot(x, w1_ref[:], preferred_element_type=f32)
    r_ref[:] = jnp.dot(x, w2_ref[:], preferred_element_type=f32)


def _row_spec():
    return pl.BlockSpec((BN, 64), lambda i: (i, 0))


def _full_spec(shape):
    nd = len(shape)
    return pl.BlockSpec(shape, lambda i: (0,) * nd)


def _embed_n(types, emb_pad, W):
    N = types.shape[0]
    types = types.reshape(N // BN, 1, BN)
    return pl.pallas_call(
        _embed_n_body,
        grid=(N // BN,),
        in_specs=[pl.BlockSpec((1, 1, BN), lambda i: (i, 0, 0)),
                  _full_spec((128, 64)), _full_spec((64, 64))],
        out_specs=[_row_spec(), _row_spec()],
        out_shape=[jax.ShapeDtypeStruct((N, 64), f32)] * 2,
    )(types, emb_pad, W)


def _embed_e(types, emb_pad, W1, W2):
    N = types.shape[0]
    types = types.reshape(N // BN, 1, BN)
    return pl.pallas_call(
        _embed_e_body,
        grid=(N // BN,),
        in_specs=[pl.BlockSpec((1, 1, BN), lambda i: (i, 0, 0)),
                  _full_spec((128, 64)), _full_spec((64, 64)),
                  _full_spec((64, 64))],
        out_specs=[_row_spec()] * 3,
        out_shape=[jax.ShapeDtypeStruct((N, 64), f32)] * 3,
    )(types, emb_pad, W1, W2)


def _proj_body(x_ref, w_ref, o_ref):
    o_ref[:] = jnp.dot(x_ref[:], w_ref[:], preferred_element_type=f32)


def _proj(x, W):
    N = x.shape[0]
    return pl.pallas_call(
        _proj_body,
        grid=(N // BN,),
        in_specs=[_row_spec(), _full_spec((64, 64))],
        out_specs=_row_spec(),
        out_shape=jax.ShapeDtypeStruct((N, 64), f32),
    )(x, W)


def _upd_n_body(x_ref, a_ref, wu_ref, wn_ref, xo_ref, po_ref):
    xn = jnp.maximum(jnp.dot(x_ref[:] + a_ref[:], wu_ref[:],
                             preferred_element_type=f32), 0.0)
    xo_ref[:] = xn
    po_ref[:] = jnp.dot(xn, wn_ref[:], preferred_element_type=f32)


def _upd_n(x, agg, Wupd, Wnext):
    N = x.shape[0]
    return pl.pallas_call(
        _upd_n_body,
        grid=(N // BN,),
        in_specs=[_row_spec(), _row_spec(), _full_spec((64, 64)),
                  _full_spec((64, 64))],
        out_specs=[_row_spec(), _row_spec()],
        out_shape=[jax.ShapeDtypeStruct((N, 64), f32)] * 2,
    )(x, agg, Wupd, Wnext)


def _upd_e_body(x_ref, a_ref, t_ref, wu_ref, wb_ref, w1_ref, w2_ref,
                xo_ref, qo_ref, ro_ref):
    w2c = jnp.dot(wb_ref[:], wu_ref[:], preferred_element_type=f32)
    xe = jnp.maximum(
        jnp.dot(x_ref[:] + a_ref[:], wu_ref[:], preferred_element_type=f32)
        + jnp.dot(t_ref[:], w2c, preferred_element_type=f32), 0.0)
    xo_ref[:] = xe
    qo_ref[:] = jnp.dot(xe, w1_ref[:], preferred_element_type=f32)
    ro_ref[:] = jnp.dot(xe, w2_ref[:], preferred_element_type=f32)


def _upd_e(x, agg, T, Wupd, Wbd, Wn1, Wn2):
    N = x.shape[0]
    return pl.pallas_call(
        _upd_e_body,
        grid=(N // BN,),
        in_specs=[_row_spec(), _row_spec(), _row_spec()]
        + [_full_spec((64, 64))] * 4,
        out_specs=[_row_spec()] * 3,
        out_shape=[jax.ShapeDtypeStruct((N, 64), f32)] * 3,
    )(x, agg, T, Wupd, Wbd, Wn1, Wn2)


def _upd_c_body(x_ref, u_ref, wu_ref, wb_ref, wn_ref, xo_ref, so_ref):
    w2c = jnp.dot(wb_ref[:], wu_ref[:], preferred_element_type=f32)
    xc = jnp.maximum(
        jnp.dot(x_ref[:], wu_ref[:], preferred_element_type=f32)
        + jnp.dot(u_ref[:], w2c, preferred_element_type=f32), 0.0)
    xo_ref[:] = xc
    so_ref[:] = jnp.dot(xc, wn_ref[:], preferred_element_type=f32)


def _upd_c(x, U, Wupd, Wbd, Wnext):
    N = x.shape[0]
    return pl.pallas_call(
        _upd_c_body,
        grid=(N // BN,),
        in_specs=[_row_spec(), _row_spec()] + [_full_spec((64, 64))] * 3,
        out_specs=[_row_spec(), _row_spec()],
        out_shape=[jax.ShapeDtypeStruct((N, 64), f32)] * 2,
    )(x, U, Wupd, Wbd, Wnext)


def _updgate_body(nblk, upd_fn, refs):
    i = pl.program_id(0)
    out_ref = refs[-1]

    @pl.when(i == nblk)
    def _zero():
        out_ref[:] = jnp.zeros((BN, 64), f32)

    @pl.when(i < nblk)
    def _compute():
        out_ref[:] = upd_fn(*refs[:-1])


def _gate_n_fn(x_ref, a_ref, r0_ref, wu_ref, wp0_ref, wp1_ref):
    xn = jnp.maximum(jnp.dot(x_ref[:] + a_ref[:], wu_ref[:],
                             preferred_element_type=f32), 0.0)
    g = jax.nn.sigmoid(jnp.dot(xn, wp0_ref[:], preferred_element_type=f32)
                       + jnp.dot(r0_ref[:], wp1_ref[:],
                                 preferred_element_type=f32))
    return g * xn


def _gate_e_fn(x_ref, a_ref, t_ref, r0_ref, wu_ref, wb_ref, wp0_ref, wp1_ref):
    w2c = jnp.dot(wb_ref[:], wu_ref[:], preferred_element_type=f32)
    xe = jnp.maximum(
        jnp.dot(x_ref[:] + a_ref[:], wu_ref[:], preferred_element_type=f32)
        + jnp.dot(t_ref[:], w2c, preferred_element_type=f32), 0.0)
    g = jax.nn.sigmoid(jnp.dot(xe, wp0_ref[:], preferred_element_type=f32)
                       + jnp.dot(r0_ref[:], wp1_ref[:],
                                 preferred_element_type=f32))
    return g * xe


def _gate_c_fn(x_ref, u_ref, r0_ref, wu_ref, wb_ref, wp0_ref, wp1_ref):
    w2c = jnp.dot(wb_ref[:], wu_ref[:], preferred_element_type=f32)
    xc = jnp.maximum(
        jnp.dot(x_ref[:], wu_ref[:], preferred_element_type=f32)
        + jnp.dot(u_ref[:], w2c, preferred_element_type=f32), 0.0)
    g = jax.nn.sigmoid(jnp.dot(xc, wp0_ref[:], preferred_element_type=f32)
                       + jnp.dot(r0_ref[:], wp1_ref[:],
                                 preferred_element_type=f32))
    return g * xc


def _updgate(fn, row_args, mat_args, N):
    nblk = N // BN
    grid = (nblk + 1,)
    clamp = lambda i: (jnp.minimum(i, nblk - 1), 0)
    in_specs = ([pl.BlockSpec((BN, 64), clamp)] * len(row_args)
                + [_full_spec((64, 64))] * len(mat_args))
    body = functools.partial(_updgate_body, nblk,
                             lambda *r: fn(*r))
    return pl.pallas_call(
        lambda *refs: body(refs),
        grid=grid,
        in_specs=in_specs,
        out_specs=_row_spec(),
        out_shape=jax.ShapeDtypeStruct((N + BN, 64), f32),
    )(*row_args, *mat_args)


def _head_body(pn_ref, pe_ref, pc_ref, w1_ref, b1_ref, w2_ref, b2_ref, o_ref):
    acc = jnp.zeros((B, OUT), f32)
    for i, p in enumerate((pn_ref, pe_ref, pc_ref)):
        h = jnp.maximum(jnp.dot(p[:], w1_ref[i], preferred_element_type=f32)
                        + b1_ref[i][None, :], 0.0)
        acc = acc + jnp.dot(h, w2_ref[pl.ds(i * 128, 128), :],
                            preferred_element_type=f32)
    o_ref[:] = acc + b2_ref[:][None, :]


def _head(pn, pe, pc, w1, b1, w2, b2):
    return pl.pallas_call(
        _head_body,
        in_specs=[pl.BlockSpec((B, 64), lambda: (0, 0))] * 3
        + [pl.BlockSpec((3, 64, 128), lambda: (0, 0, 0)),
           pl.BlockSpec((3, 128), lambda: (0, 0)),
           pl.BlockSpec((384, OUT), lambda: (0, 0)),
           pl.BlockSpec((OUT,), lambda: (0,))],
        out_specs=pl.BlockSpec((B, OUT), lambda: (0, 0)),
        out_shape=jax.ShapeDtypeStruct((B, OUT), f32),
    )(pn, pe, pc, w1, b1, w2, b2)


def kernel(atom_type, bond_type, n_up_i, n_up_j, n_up_attr_idx,
           e_up_i, e_up_j, e_up_attr_idx, e_bd_cell, e_bd_node,
           c_bd_cell, c_bd_edge, n_batch, n_pos, e_batch, e_pos,
           c_batch, c_pos, atom_emb, bond_emb, Wn_up, Wn_upd,
           We_up, We_bd, We_upd, Wc_bd, Wc_upd, Wpool,
           lin1_w, lin1_b, lin2_w, lin2_b):
    atom_pad = jnp.pad(atom_emb, ((0, 128 - atom_emb.shape[0]), (0, 0)))
    bond_pad = jnp.pad(bond_emb, ((0, 128 - bond_emb.shape[0]), (0, 0)))

    xn, Pn = _embed_n(atom_type, atom_pad, Wn_up[0])
    xe, Qe, Re = _embed_e(bond_type, bond_pad, Wn_up[0], We_up[0])
    xc = _edge_pass1(c_bd_cell, c_bd_edge, xe, NC)
    Sc = _proj(xc, We_up[0])
    xn0, xe0, xc0 = xn, xe, xc

    gated = [None, None, None]
    for l in range(L):
        agg_n = _edge_pass2(n_up_i, n_up_j, n_up_attr_idx, Pn, Qe, NN)
        agg_e = _edge_pass2(e_up_i, e_up_j, e_up_attr_idx, Re, Sc, NE)
        T = _edge_pass1(e_bd_cell, e_bd_node, xn, NE)
        U = _edge_pass1(c_bd_cell, c_bd_edge, xe, NC)
        if l < L - 1:
            xn, Pn = _upd_n(xn, agg_n, Wn_upd[l], Wn_up[l + 1])
            xe, Qe, Re = _upd_e(xe, agg_e, T, We_upd[l], We_bd[l],
                                Wn_up[l + 1], We_up[l + 1])
            xc, Sc = _upd_c(xc, U, Wc_upd[l], Wc_bd[l], We_up[l + 1])
        else:
            gated[0] = _updgate(_gate_n_fn, (xn, agg_n, xn0),
                                (Wn_upd[l], Wpool[0, 0], Wpool[0, 1]), NN)
            gated[1] = _updgate(_gate_e_fn, (xe, agg_e, T, xe0),
                                (We_upd[l], We_bd[l], Wpool[1, 0],
                                 Wpool[1, 1]), NE)
            gated[2] = _updgate(_gate_c_fn, (xc, U, xc0),
                                (Wc_upd[l], Wc_bd[l], Wpool[2, 0],
                                 Wpool[2, 1]), NC)

    pn = _pool_pass(n_batch, n_pos, gated[0], NN, MAXN)
    pe = _pool_pass(e_batch, e_pos, gated[1], NE, MAXE)
    pc = _pool_pass(c_batch, c_pos, gated[2], NC, MAXC)
    return _head(pn, pe, pc, lin1_w, lin1_b, lin2_w, lin2_b)
